# baseline (speedup 1.0000x reference)
import functools

import jax
import jax.numpy as jnp
from jax import lax
from jax.experimental import pallas as pl
from jax.experimental.pallas import tpu as pltpu
from jax.experimental.pallas import tpu_sc as plsc

_N1 = 8000
_N0 = 4000
_E = 320000
_D_IN = 128
_D_H = 128
_D_OUT = 64
_EPS = 1e-5
_NW = 32
_CH = 128
_IB = 16
_NB = 4
_PTRBUF = 264


def _make_seg_sum(num_nodes, d):
  octets = num_nodes // 8
  base_oct = octets // _NW
  extra = octets % _NW
  nn_lo = 8 * base_oct
  nn_max = nn_lo + 8
  rg = ((nn_max + 1 + _CH - 1) // _CH) * _CH
  steps = tuple(s for s in (256, 128, 64, 32, 16, 8, 4, 2, 1) if s <= nn_max)
  mesh = plsc.VectorSubcoreMesh(
      core_axis_name="c", subcore_axis_name="s", num_cores=2, num_subcores=16)

  @functools.partial(
      pl.kernel,
      out_type=jax.ShapeDtypeStruct((num_nodes, d), jnp.float32),
      mesh=mesh,
      scratch_types=[
          pltpu.VMEM((_PTRBUF,), jnp.int32),
          pltpu.VMEM((_IB, _CH), jnp.int32),
      ] + [pltpu.VMEM((_CH,), jnp.int32)] * _NB
        + [pltpu.VMEM((_CH, d), jnp.float32)] * _NB
        + [pltpu.VMEM_SHARED((rg * 16, d), jnp.float32)]
        + [pltpu.SemaphoreType.DMA] * (2 * _NB),
      compiler_params=pltpu.CompilerParams(
          needs_layout_passes=False, use_tc_tiling_on_sc=False),
  )
  def seg_sum(table_hbm, ptr_hbm, idx2_hbm, out_hbm, ptr_v, idxb_v, *rest):
    segc = rest[:_NB]
    rows = rest[_NB:2 * _NB]
    acc_sh = rest[2 * _NB]
    semg = rest[2 * _NB + 1:3 * _NB + 1]
    sems = rest[3 * _NB + 1:4 * _NB + 1]
    rows_v0 = rows[0]
    semg0 = semg[0]
    sid = lax.axis_index("s")
    wid = lax.axis_index("c") * 16 + sid
    rbase = sid * rg
    is_big = wid < extra
    nn = jnp.where(is_big, nn_max, nn_lo)
    nlo = pl.multiple_of(8 * (wid * base_oct + jnp.minimum(wid, extra)), 8)
    pltpu.sync_copy(ptr_hbm.at[pl.ds(nlo, _PTRBUF)], ptr_v)

    def zrow(r, carry):
      for k in range(d // 16):
        rows_v0[r, pl.ds(k * 16, 16)] = jnp.zeros((16,), jnp.float32)
      return carry
    lax.fori_loop(0, _CH, zrow, 0)
    for j in range(rg // _CH):
      pltpu.sync_copy(rows_v0, acc_sh.at[pl.ds(rbase + j * _CH, _CH)])

    elo = jnp.min(plsc.load_gather(ptr_v, [jnp.zeros((16,), jnp.int32)]))
    ehi = jnp.min(plsc.load_gather(ptr_v, [jnp.full((16,), nn, jnp.int32)]))
    c0 = lax.div(elo, _CH)
    c1 = lax.div(ehi + (_CH - 1), _CH)
    nch = c1 - c0
    ngrp = lax.div(nch + (_IB - 1), _IB)
    lanes = jnp.arange(16, dtype=jnp.int32)

    def seg_ids(c, out_ref):
      ebase = c * _CH
      for g in range(_CH // 16):
        e = ebase + g * 16 + lanes
        res = jnp.zeros((16,), jnp.int32)
        for stv in steps:
          cand = res + stv
          candc = jnp.minimum(cand, nn)
          pv = plsc.load_gather(ptr_v, [candc])
          ok = (cand <= nn) & (pv <= e)
          res = jnp.where(ok, cand, res)
        seg = rbase + jnp.where((e >= elo) & (e < ehi), res, nn_max)
        out_ref[pl.ds(g * 16, 16)] = seg

    def wait_scatter(b):
      pltpu.make_async_copy(rows[b], acc_sh.at[segc[b]], sems[b]).wait()

    nblk = idx2_hbm.shape[0]

    def grp(kb, carry):
      cb = c0 + kb * _IB
      cload = jnp.minimum(cb, nblk - _IB)
      sh = cb - cload
      pltpu.sync_copy(idx2_hbm.at[pl.ds(cload, _IB)], idxb_v)
      pltpu.async_copy(table_hbm.at[idxb_v.at[sh]], rows_v0, semg0)

      @pl.when(cb + 1 < c1)
      def _():
        pltpu.async_copy(table_hbm.at[idxb_v.at[1 + sh]], rows[1], semg[1])

      def quad(p, carry2):
        j0 = _NB * p
        for b in range(_NB):
          j = j0 + b
          c = cb + j
          bn = (b + 2) % _NB

          @pl.when((c - 2 >= c0) & (c - 2 < c1))
          def _(bn=bn):
            wait_scatter(bn)

          @pl.when((j + 2 < _IB) & (c + 2 < c1))
          def _(j=j, bn=bn):
            pltpu.async_copy(
                table_hbm.at[idxb_v.at[j + 2 + sh]], rows[bn], semg[bn])

          @pl.when(c < c1)
          def _(c=c, j=j, b=b):
            seg_ids(c, segc[b])
            pltpu.make_async_copy(
                table_hbm.at[idxb_v.at[j + sh]], rows[b], semg[b]).wait()
            pltpu.async_copy(rows[b], acc_sh.at[segc[b]], sems[b], add=True)
        return carry2
      lax.fori_loop(0, _IB // _NB, quad, 0)
      return carry
    lax.fori_loop(0, ngrp, grp, 0)

    t_end = c0 + ngrp * _IB

    @pl.when((t_end - 2 >= c0) & (t_end - 2 < c1))
    def _():
      wait_scatter(2)

    @pl.when((t_end - 1 >= c0) & (t_end - 1 < c1))
    def _():
      wait_scatter(3)

    pltpu.sync_copy(acc_sh.at[pl.ds(rbase, nn_lo)],
                    out_hbm.at[pl.ds(nlo, nn_lo)])

    @pl.when(is_big)
    def _():
      pltpu.sync_copy(acc_sh.at[pl.ds(rbase + nn_lo, 8)],
                      out_hbm.at[pl.ds(nlo + nn_lo, 8)])

  return seg_sum


_seg_sum_cache = {}


def _seg_sum(num_nodes, d):
  key = (num_nodes, d)
  if key not in _seg_sum_cache:
    _seg_sum_cache[key] = _make_seg_sum(num_nodes, d)
  return _seg_sum_cache[key]


def _mm_t(a, w):
  return lax.dot_general(a, w, (((1,), (1,)), ((), ())),
                         preferred_element_type=jnp.float32)


def _layer0_tc(agg0, dr0, x8, wl0, wr0, b0, scale, shift, wl1):
  bl = 800
  grid = _N1 // bl

  def body(agg_r, dr_r, x_r, wl0_r, wr0_r, b0_r, sc_r, sh_r, wl1_r,
           h_r, hw_r):
    z = _mm_t(agg_r[...] * dr_r[...], wl0_r[...])
    z = z + _mm_t(x_r[...], wr0_r[...])
    z = z + b0_r[...]
    h = jnp.maximum(z * sc_r[...] + sh_r[...], 0.0)
    h_r[...] = h
    hw_r[...] = _mm_t(h, wl1_r[...])

  full = lambda shape: pl.BlockSpec(shape, lambda i: (0, 0))
  return pl.pallas_call(
      body,
      grid=(grid,),
      in_specs=[
          pl.BlockSpec((bl, _D_H), lambda i: (i, 0)),
          pl.BlockSpec((bl, 1), lambda i: (i, 0)),
          pl.BlockSpec((bl, _D_IN), lambda i: (i, 0)),
          full((_D_H, _D_IN)),
          full((_D_H, _D_IN)),
          full((1, _D_H)),
          full((1, _D_H)),
          full((1, _D_H)),
          full((_D_OUT, _D_H)),
      ],
      out_specs=[
          pl.BlockSpec((bl, _D_H), lambda i: (i, 0)),
          pl.BlockSpec((bl, _D_OUT), lambda i: (i, 0)),
      ],
      out_shape=[
          jax.ShapeDtypeStruct((_N1, _D_H), jnp.float32),
          jax.ShapeDtypeStruct((_N1, _D_OUT), jnp.float32),
      ],
  )(agg0, dr0, x8, wl0, wr0, b0, scale, shift, wl1)


def _layer1_tc(agg1, dr1, h4, wr1, b1):
  bl = 400
  grid = _N0 // bl

  def body(agg_r, dr_r, h_r, wr1_r, b1_r, o_r):
    z = agg_r[...] * dr_r[...]
    z = z + _mm_t(h_r[...], wr1_r[...])
    z = z + b1_r[...]
    m = jnp.max(z, axis=-1, keepdims=True)
    ez = jnp.exp(z - m)
    o_r[...] = z - m - jnp.log(jnp.sum(ez, axis=-1, keepdims=True))

  full = lambda shape: pl.BlockSpec(shape, lambda i: (0, 0))
  return pl.pallas_call(
      body,
      grid=(grid,),
      in_specs=[
          pl.BlockSpec((bl, _D_OUT), lambda i: (i, 0)),
          pl.BlockSpec((bl, 1), lambda i: (i, 0)),
          pl.BlockSpec((bl, _D_H), lambda i: (i, 0)),
          full((_D_OUT, _D_H)),
          full((1, _D_OUT)),
      ],
      out_specs=pl.BlockSpec((bl, _D_OUT), lambda i: (i, 0)),
      out_shape=jax.ShapeDtypeStruct((_N0, _D_OUT), jnp.float32),
  )(agg1, dr1, h4, wr1, b1)


def kernel(x, ptr, idx, W_l0, b_l0, W_r0, gamma, beta, rmean, rvar,
           W_l1, b_l1, W_r1):
  ptr_pad = jnp.concatenate(
      [ptr, jnp.full((_PTRBUF,), jnp.iinfo(jnp.int32).max, jnp.int32)])
  idx2 = idx.reshape(_E // _CH, _CH)

  dr0 = 1.0 / jnp.maximum(ptr[1:_N1 + 1] - ptr[:_N1], 1).astype(jnp.float32)
  dr1 = 1.0 / jnp.maximum(ptr[1:_N0 + 1] - ptr[:_N0], 1).astype(jnp.float32)

  scale = gamma * lax.rsqrt(rvar + _EPS)
  shift = beta - rmean * scale

  agg0 = _seg_sum(_N1, _D_H)(x, ptr_pad, idx2)
  h, hw = _layer0_tc(agg0, dr0[:, None], x[:_N1], W_l0, W_r0,
                     b_l0[None, :], scale[None, :], shift[None, :], W_l1)
  agg1 = _seg_sum(_N0, _D_OUT)(hw, ptr_pad, idx2)
  return _layer1_tc(agg1, dr1[:, None], h[:_N0], W_r1, b_l1[None, :])


# speedup vs baseline: 1.0828x; 1.0828x over previous
import functools

import jax
import jax.numpy as jnp
from jax import lax
from jax.experimental import pallas as pl
from jax.experimental.pallas import tpu as pltpu
from jax.experimental.pallas import tpu_sc as plsc

_N1 = 8000
_N0 = 4000
_E = 320000
_D_IN = 128
_D_H = 128
_D_OUT = 64
_EPS = 1e-5
_NW = 32
_CH = 128
_IB = 15
_NB = 3
_PTRBUF = 264


def _make_seg_sum(num_nodes, d):
  octets = num_nodes // 8
  base_oct = octets // _NW
  extra = octets % _NW
  nn_lo = 8 * base_oct
  nn_max = nn_lo + 8
  rg = ((nn_max + 1 + _CH - 1) // _CH) * _CH
  steps = tuple(s for s in (256, 128, 64, 32, 16, 8, 4, 2, 1) if s <= nn_max)
  mesh = plsc.VectorSubcoreMesh(
      core_axis_name="c", subcore_axis_name="s", num_cores=2, num_subcores=16)

  @functools.partial(
      pl.kernel,
      out_type=jax.ShapeDtypeStruct((num_nodes, d), jnp.float32),
      mesh=mesh,
      scratch_types=[
          pltpu.VMEM((_PTRBUF,), jnp.int32),
          pltpu.VMEM((_IB, _CH), jnp.int32),
      ] + [pltpu.VMEM((_CH,), jnp.int32)] * _NB
        + [pltpu.VMEM((_CH, d), jnp.float32)] * _NB
        + [pltpu.VMEM_SHARED((rg * 16, d), jnp.float32)]
        + [pltpu.SemaphoreType.DMA] * (2 * _NB),
      compiler_params=pltpu.CompilerParams(
          needs_layout_passes=False, use_tc_tiling_on_sc=False),
  )
  def seg_sum(table_hbm, ptr_hbm, idx2_hbm, out_hbm, ptr_v, idxb_v, *rest):
    segc = rest[:_NB]
    rows = rest[_NB:2 * _NB]
    acc_sh = rest[2 * _NB]
    semg = rest[2 * _NB + 1:3 * _NB + 1]
    sems = rest[3 * _NB + 1:4 * _NB + 1]
    rows_v0 = rows[0]
    semg0 = semg[0]
    sid = lax.axis_index("s")
    wid = lax.axis_index("c") * 16 + sid
    rbase = sid * rg
    is_big = wid < extra
    nn = jnp.where(is_big, nn_max, nn_lo)
    nlo = pl.multiple_of(8 * (wid * base_oct + jnp.minimum(wid, extra)), 8)
    pltpu.sync_copy(ptr_hbm.at[pl.ds(nlo, _PTRBUF)], ptr_v)

    def zrow(r, carry):
      for k in range(d // 16):
        rows_v0[r, pl.ds(k * 16, 16)] = jnp.zeros((16,), jnp.float32)
      return carry
    lax.fori_loop(0, _CH, zrow, 0)
    for j in range(rg // _CH):
      pltpu.sync_copy(rows_v0, acc_sh.at[pl.ds(rbase + j * _CH, _CH)])

    elo = jnp.min(plsc.load_gather(ptr_v, [jnp.zeros((16,), jnp.int32)]))
    ehi = jnp.min(plsc.load_gather(ptr_v, [jnp.full((16,), nn, jnp.int32)]))
    c0 = lax.div(elo, _CH)
    c1 = lax.div(ehi + (_CH - 1), _CH)
    nch = c1 - c0
    ngrp = lax.div(nch + (_IB - 1), _IB)
    lanes = jnp.arange(16, dtype=jnp.int32)

    def seg_ids(c, out_ref):
      ebase = c * _CH
      for g in range(_CH // 16):
        e = ebase + g * 16 + lanes
        res = jnp.zeros((16,), jnp.int32)
        for stv in steps:
          cand = res + stv
          candc = jnp.minimum(cand, nn)
          pv = plsc.load_gather(ptr_v, [candc])
          ok = (cand <= nn) & (pv <= e)
          res = jnp.where(ok, cand, res)
        seg = rbase + jnp.where((e >= elo) & (e < ehi), res, nn_max)
        out_ref[pl.ds(g * 16, 16)] = seg

    def wait_scatter(b):
      pltpu.make_async_copy(rows[b], acc_sh.at[segc[b]], sems[b]).wait()

    nblk = idx2_hbm.shape[0]

    def grp(kb, carry):
      cb = c0 + kb * _IB
      cload = jnp.minimum(cb, nblk - _IB)
      sh = cb - cload
      pltpu.sync_copy(idx2_hbm.at[pl.ds(cload, _IB)], idxb_v)
      pltpu.async_copy(table_hbm.at[idxb_v.at[sh]], rows_v0, semg0)

      def triple(p, carry2):
        j0 = _NB * p
        for b in range(_NB):
          j = j0 + b
          c = cb + j
          bn = (b + 1) % _NB

          @pl.when((c - 2 >= c0) & (c - 2 < c1))
          def _(bn=bn):
            wait_scatter(bn)

          @pl.when((j + 1 < _IB) & (c + 1 < c1))
          def _(j=j, bn=bn):
            pltpu.async_copy(
                table_hbm.at[idxb_v.at[j + 1 + sh]], rows[bn], semg[bn])

          @pl.when(c < c1)
          def _(c=c, j=j, b=b):
            seg_ids(c, segc[b])
            pltpu.make_async_copy(
                table_hbm.at[idxb_v.at[j + sh]], rows[b], semg[b]).wait()
            pltpu.async_copy(rows[b], acc_sh.at[segc[b]], sems[b], add=True)
        return carry2
      lax.fori_loop(0, _IB // _NB, triple, 0)
      return carry
    lax.fori_loop(0, ngrp, grp, 0)

    t_end = c0 + ngrp * _IB

    @pl.when((t_end - 2 >= c0) & (t_end - 2 < c1))
    def _():
      wait_scatter(1)

    @pl.when((t_end - 1 >= c0) & (t_end - 1 < c1))
    def _():
      wait_scatter(2)

    pltpu.sync_copy(acc_sh.at[pl.ds(rbase, nn_lo)],
                    out_hbm.at[pl.ds(nlo, nn_lo)])

    @pl.when(is_big)
    def _():
      pltpu.sync_copy(acc_sh.at[pl.ds(rbase + nn_lo, 8)],
                      out_hbm.at[pl.ds(nlo + nn_lo, 8)])

  return seg_sum


_seg_sum_cache = {}


def _seg_sum(num_nodes, d):
  key = (num_nodes, d)
  if key not in _seg_sum_cache:
    _seg_sum_cache[key] = _make_seg_sum(num_nodes, d)
  return _seg_sum_cache[key]


def _mm_t(a, w):
  return lax.dot_general(a, w, (((1,), (1,)), ((), ())),
                         preferred_element_type=jnp.float32)


def _layer0_tc(agg0, dr0, x8, wl0, wr0, b0, scale, shift, wl1):
  bl = 800
  grid = _N1 // bl

  def body(agg_r, dr_r, x_r, wl0_r, wr0_r, b0_r, sc_r, sh_r, wl1_r,
           h_r, hw_r):
    z = _mm_t(agg_r[...] * dr_r[...], wl0_r[...])
    z = z + _mm_t(x_r[...], wr0_r[...])
    z = z + b0_r[...]
    h = jnp.maximum(z * sc_r[...] + sh_r[...], 0.0)
    h_r[...] = h
    hw_r[...] = _mm_t(h, wl1_r[...])

  full = lambda shape: pl.BlockSpec(shape, lambda i: (0, 0))
  return pl.pallas_call(
      body,
      grid=(grid,),
      in_specs=[
          pl.BlockSpec((bl, _D_H), lambda i: (i, 0)),
          pl.BlockSpec((bl, 1), lambda i: (i, 0)),
          pl.BlockSpec((bl, _D_IN), lambda i: (i, 0)),
          full((_D_H, _D_IN)),
          full((_D_H, _D_IN)),
          full((1, _D_H)),
          full((1, _D_H)),
          full((1, _D_H)),
          full((_D_OUT, _D_H)),
      ],
      out_specs=[
          pl.BlockSpec((bl, _D_H), lambda i: (i, 0)),
          pl.BlockSpec((bl, _D_OUT), lambda i: (i, 0)),
      ],
      out_shape=[
          jax.ShapeDtypeStruct((_N1, _D_H), jnp.float32),
          jax.ShapeDtypeStruct((_N1, _D_OUT), jnp.float32),
      ],
  )(agg0, dr0, x8, wl0, wr0, b0, scale, shift, wl1)


def _layer1_tc(agg1, dr1, h4, wr1, b1):
  bl = 400
  grid = _N0 // bl

  def body(agg_r, dr_r, h_r, wr1_r, b1_r, o_r):
    z = agg_r[...] * dr_r[...]
    z = z + _mm_t(h_r[...], wr1_r[...])
    z = z + b1_r[...]
    m = jnp.max(z, axis=-1, keepdims=True)
    ez = jnp.exp(z - m)
    o_r[...] = z - m - jnp.log(jnp.sum(ez, axis=-1, keepdims=True))

  full = lambda shape: pl.BlockSpec(shape, lambda i: (0, 0))
  return pl.pallas_call(
      body,
      grid=(grid,),
      in_specs=[
          pl.BlockSpec((bl, _D_OUT), lambda i: (i, 0)),
          pl.BlockSpec((bl, 1), lambda i: (i, 0)),
          pl.BlockSpec((bl, _D_H), lambda i: (i, 0)),
          full((_D_OUT, _D_H)),
          full((1, _D_OUT)),
      ],
      out_specs=pl.BlockSpec((bl, _D_OUT), lambda i: (i, 0)),
      out_shape=jax.ShapeDtypeStruct((_N0, _D_OUT), jnp.float32),
  )(agg1, dr1, h4, wr1, b1)


def kernel(x, ptr, idx, W_l0, b_l0, W_r0, gamma, beta, rmean, rvar,
           W_l1, b_l1, W_r1):
  ptr_pad = jnp.concatenate(
      [ptr, jnp.full((_PTRBUF,), jnp.iinfo(jnp.int32).max, jnp.int32)])
  idx2 = idx.reshape(_E // _CH, _CH)

  dr0 = 1.0 / jnp.maximum(ptr[1:_N1 + 1] - ptr[:_N1], 1).astype(jnp.float32)
  dr1 = 1.0 / jnp.maximum(ptr[1:_N0 + 1] - ptr[:_N0], 1).astype(jnp.float32)

  scale = gamma * lax.rsqrt(rvar + _EPS)
  shift = beta - rmean * scale

  agg0 = _seg_sum(_N1, _D_H)(x, ptr_pad, idx2)
  h, hw = _layer0_tc(agg0, dr0[:, None], x[:_N1], W_l0, W_r0,
                     b_l0[None, :], scale[None, :], shift[None, :], W_l1)
  agg1 = _seg_sum(_N0, _D_OUT)(hw, ptr_pad, idx2)
  return _layer1_tc(agg1, dr1[:, None], h[:_N0], W_r1, b_l1[None, :])


# speedup vs baseline: 1.1107x; 1.0258x over previous
import functools

import jax
import jax.numpy as jnp
from jax import lax
from jax.experimental import pallas as pl
from jax.experimental.pallas import tpu as pltpu
from jax.experimental.pallas import tpu_sc as plsc

_N1 = 8000
_N0 = 4000
_E = 320000
_D_IN = 128
_D_H = 128
_D_OUT = 64
_EPS = 1e-5
_NW = 32
_CH = 128
_IB = 30
_NB = 3
_PTRBUF = 264


def _make_seg_sum(num_nodes, d):
  octets = num_nodes // 8
  base_oct = octets // _NW
  extra = octets % _NW
  nn_lo = 8 * base_oct
  nn_max = nn_lo + 8
  rg = ((nn_max + 1 + _CH - 1) // _CH) * _CH
  steps = tuple(s for s in (256, 128, 64, 32, 16, 8, 4, 2, 1) if s <= nn_max)
  mesh = plsc.VectorSubcoreMesh(
      core_axis_name="c", subcore_axis_name="s", num_cores=2, num_subcores=16)

  @functools.partial(
      pl.kernel,
      out_type=jax.ShapeDtypeStruct((num_nodes, d), jnp.float32),
      mesh=mesh,
      scratch_types=[
          pltpu.VMEM((_PTRBUF,), jnp.int32),
          pltpu.VMEM((_IB, _CH), jnp.int32),
      ] + [pltpu.VMEM((_CH,), jnp.int32)] * _NB
        + [pltpu.VMEM((_CH, d), jnp.float32)] * _NB
        + [pltpu.VMEM_SHARED((rg * 16, d), jnp.float32)]
        + [pltpu.SemaphoreType.DMA] * (2 * _NB),
      compiler_params=pltpu.CompilerParams(
          needs_layout_passes=False, use_tc_tiling_on_sc=False),
  )
  def seg_sum(table_hbm, ptr_hbm, idx2_hbm, out_hbm, ptr_v, idxb_v, *rest):
    segc = rest[:_NB]
    rows = rest[_NB:2 * _NB]
    acc_sh = rest[2 * _NB]
    semg = rest[2 * _NB + 1:3 * _NB + 1]
    sems = rest[3 * _NB + 1:4 * _NB + 1]
    rows_v0 = rows[0]
    semg0 = semg[0]
    sid = lax.axis_index("s")
    wid = lax.axis_index("c") * 16 + sid
    rbase = sid * rg
    is_big = wid < extra
    nn = jnp.where(is_big, nn_max, nn_lo)
    nlo = pl.multiple_of(8 * (wid * base_oct + jnp.minimum(wid, extra)), 8)
    pltpu.sync_copy(ptr_hbm.at[pl.ds(nlo, _PTRBUF)], ptr_v)

    def zrow(r, carry):
      for k in range(d // 16):
        rows_v0[r, pl.ds(k * 16, 16)] = jnp.zeros((16,), jnp.float32)
      return carry
    lax.fori_loop(0, _CH, zrow, 0)
    for j in range(rg // _CH):
      pltpu.sync_copy(rows_v0, acc_sh.at[pl.ds(rbase + j * _CH, _CH)])

    elo = jnp.min(plsc.load_gather(ptr_v, [jnp.zeros((16,), jnp.int32)]))
    ehi = jnp.min(plsc.load_gather(ptr_v, [jnp.full((16,), nn, jnp.int32)]))
    c0 = lax.div(elo, _CH)
    c1 = lax.div(ehi + (_CH - 1), _CH)
    nch = c1 - c0
    ngrp = lax.div(nch + (_IB - 1), _IB)
    lanes = jnp.arange(16, dtype=jnp.int32)

    def seg_ids(c, out_ref):
      ebase = c * _CH
      for g in range(_CH // 16):
        e = ebase + g * 16 + lanes
        res = jnp.zeros((16,), jnp.int32)
        for stv in steps:
          cand = res + stv
          candc = jnp.minimum(cand, nn)
          pv = plsc.load_gather(ptr_v, [candc])
          ok = (cand <= nn) & (pv <= e)
          res = jnp.where(ok, cand, res)
        seg = rbase + jnp.where((e >= elo) & (e < ehi), res, nn_max)
        out_ref[pl.ds(g * 16, 16)] = seg

    def wait_scatter(b):
      pltpu.make_async_copy(rows[b], acc_sh.at[segc[b]], sems[b]).wait()

    nblk = idx2_hbm.shape[0]

    def grp(kb, carry):
      cb = c0 + kb * _IB
      cload = jnp.minimum(cb, nblk - _IB)
      sh = cb - cload
      pltpu.sync_copy(idx2_hbm.at[pl.ds(cload, _IB)], idxb_v)
      pltpu.async_copy(table_hbm.at[idxb_v.at[sh]], rows_v0, semg0)

      def triple(p, carry2):
        j0 = _NB * p
        for b in range(_NB):
          j = j0 + b
          c = cb + j
          bn = (b + 1) % _NB

          @pl.when((c - 2 >= c0) & (c - 2 < c1))
          def _(bn=bn):
            wait_scatter(bn)

          @pl.when((j + 1 < _IB) & (c + 1 < c1))
          def _(j=j, bn=bn):
            pltpu.async_copy(
                table_hbm.at[idxb_v.at[j + 1 + sh]], rows[bn], semg[bn])

          @pl.when(c < c1)
          def _(c=c, j=j, b=b):
            seg_ids(c, segc[b])
            pltpu.make_async_copy(
                table_hbm.at[idxb_v.at[j + sh]], rows[b], semg[b]).wait()
            pltpu.async_copy(rows[b], acc_sh.at[segc[b]], sems[b], add=True)
        return carry2
      lax.fori_loop(0, _IB // _NB, triple, 0)
      return carry
    lax.fori_loop(0, ngrp, grp, 0)

    t_end = c0 + ngrp * _IB

    @pl.when((t_end - 2 >= c0) & (t_end - 2 < c1))
    def _():
      wait_scatter(1)

    @pl.when((t_end - 1 >= c0) & (t_end - 1 < c1))
    def _():
      wait_scatter(2)

    pltpu.sync_copy(acc_sh.at[pl.ds(rbase, nn_lo)],
                    out_hbm.at[pl.ds(nlo, nn_lo)])

    @pl.when(is_big)
    def _():
      pltpu.sync_copy(acc_sh.at[pl.ds(rbase + nn_lo, 8)],
                      out_hbm.at[pl.ds(nlo + nn_lo, 8)])

  return seg_sum


_seg_sum_cache = {}


def _seg_sum(num_nodes, d):
  key = (num_nodes, d)
  if key not in _seg_sum_cache:
    _seg_sum_cache[key] = _make_seg_sum(num_nodes, d)
  return _seg_sum_cache[key]


def _mm_t(a, w):
  return lax.dot_general(a, w, (((1,), (1,)), ((), ())),
                         preferred_element_type=jnp.float32)


def _layer0_tc(agg0, dr0, x8, wl0, wr0, b0, scale, shift, wl1):
  bl = 800
  grid = _N1 // bl

  def body(agg_r, dr_r, x_r, wl0_r, wr0_r, b0_r, sc_r, sh_r, wl1_r,
           h_r, hw_r):
    z = _mm_t(agg_r[...] * dr_r[...], wl0_r[...])
    z = z + _mm_t(x_r[...], wr0_r[...])
    z = z + b0_r[...]
    h = jnp.maximum(z * sc_r[...] + sh_r[...], 0.0)
    h_r[...] = h
    hw_r[...] = _mm_t(h, wl1_r[...])

  full = lambda shape: pl.BlockSpec(shape, lambda i: (0, 0))
  return pl.pallas_call(
      body,
      grid=(grid,),
      in_specs=[
          pl.BlockSpec((bl, _D_H), lambda i: (i, 0)),
          pl.BlockSpec((bl, 1), lambda i: (i, 0)),
          pl.BlockSpec((bl, _D_IN), lambda i: (i, 0)),
          full((_D_H, _D_IN)),
          full((_D_H, _D_IN)),
          full((1, _D_H)),
          full((1, _D_H)),
          full((1, _D_H)),
          full((_D_OUT, _D_H)),
      ],
      out_specs=[
          pl.BlockSpec((bl, _D_H), lambda i: (i, 0)),
          pl.BlockSpec((bl, _D_OUT), lambda i: (i, 0)),
      ],
      out_shape=[
          jax.ShapeDtypeStruct((_N1, _D_H), jnp.float32),
          jax.ShapeDtypeStruct((_N1, _D_OUT), jnp.float32),
      ],
  )(agg0, dr0, x8, wl0, wr0, b0, scale, shift, wl1)


def _layer1_tc(agg1, dr1, h4, wr1, b1):
  bl = 400
  grid = _N0 // bl

  def body(agg_r, dr_r, h_r, wr1_r, b1_r, o_r):
    z = agg_r[...] * dr_r[...]
    z = z + _mm_t(h_r[...], wr1_r[...])
    z = z + b1_r[...]
    m = jnp.max(z, axis=-1, keepdims=True)
    ez = jnp.exp(z - m)
    o_r[...] = z - m - jnp.log(jnp.sum(ez, axis=-1, keepdims=True))

  full = lambda shape: pl.BlockSpec(shape, lambda i: (0, 0))
  return pl.pallas_call(
      body,
      grid=(grid,),
      in_specs=[
          pl.BlockSpec((bl, _D_OUT), lambda i: (i, 0)),
          pl.BlockSpec((bl, 1), lambda i: (i, 0)),
          pl.BlockSpec((bl, _D_H), lambda i: (i, 0)),
          full((_D_OUT, _D_H)),
          full((1, _D_OUT)),
      ],
      out_specs=pl.BlockSpec((bl, _D_OUT), lambda i: (i, 0)),
      out_shape=jax.ShapeDtypeStruct((_N0, _D_OUT), jnp.float32),
  )(agg1, dr1, h4, wr1, b1)


def kernel(x, ptr, idx, W_l0, b_l0, W_r0, gamma, beta, rmean, rvar,
           W_l1, b_l1, W_r1):
  ptr_pad = jnp.concatenate(
      [ptr, jnp.full((_PTRBUF,), jnp.iinfo(jnp.int32).max, jnp.int32)])
  idx2 = idx.reshape(_E // _CH, _CH)

  dr0 = 1.0 / jnp.maximum(ptr[1:_N1 + 1] - ptr[:_N1], 1).astype(jnp.float32)
  dr1 = 1.0 / jnp.maximum(ptr[1:_N0 + 1] - ptr[:_N0], 1).astype(jnp.float32)

  scale = gamma * lax.rsqrt(rvar + _EPS)
  shift = beta - rmean * scale

  agg0 = _seg_sum(_N1, _D_H)(x, ptr_pad, idx2)
  h, hw = _layer0_tc(agg0, dr0[:, None], x[:_N1], W_l0, W_r0,
                     b_l0[None, :], scale[None, :], shift[None, :], W_l1)
  agg1 = _seg_sum(_N0, _D_OUT)(hw, ptr_pad, idx2)
  return _layer1_tc(agg1, dr1[:, None], h[:_N0], W_r1, b_l1[None, :])


# speedup vs baseline: 1.1224x; 1.0106x over previous
import functools

import jax
import jax.numpy as jnp
from jax import lax
from jax.experimental import pallas as pl
from jax.experimental.pallas import tpu as pltpu
from jax.experimental.pallas import tpu_sc as plsc

_N1 = 8000
_N0 = 4000
_E = 320000
_D_IN = 128
_D_H = 128
_D_OUT = 64
_EPS = 1e-5
_NW = 32
_CH = 128
_IB = 60
_NB = 3
_PTRBUF = 264


def _make_seg_sum(num_nodes, d):
  octets = num_nodes // 8
  base_oct = octets // _NW
  extra = octets % _NW
  nn_lo = 8 * base_oct
  nn_max = nn_lo + 8
  rg = ((nn_max + 1 + _CH - 1) // _CH) * _CH
  steps = tuple(s for s in (256, 128, 64, 32, 16, 8, 4, 2, 1) if s <= nn_max)
  mesh = plsc.VectorSubcoreMesh(
      core_axis_name="c", subcore_axis_name="s", num_cores=2, num_subcores=16)

  @functools.partial(
      pl.kernel,
      out_type=jax.ShapeDtypeStruct((num_nodes, d), jnp.float32),
      mesh=mesh,
      scratch_types=[
          pltpu.VMEM((_PTRBUF,), jnp.int32),
          pltpu.VMEM((_IB, _CH), jnp.int32),
      ] + [pltpu.VMEM((_CH,), jnp.int32)] * _NB
        + [pltpu.VMEM((_CH, d), jnp.float32)] * _NB
        + [pltpu.VMEM_SHARED((rg * 16, d), jnp.float32)]
        + [pltpu.SemaphoreType.DMA] * (2 * _NB),
      compiler_params=pltpu.CompilerParams(
          needs_layout_passes=False, use_tc_tiling_on_sc=False),
  )
  def seg_sum(table_hbm, ptr_hbm, idx2_hbm, out_hbm, ptr_v, idxb_v, *rest):
    segc = rest[:_NB]
    rows = rest[_NB:2 * _NB]
    acc_sh = rest[2 * _NB]
    semg = rest[2 * _NB + 1:3 * _NB + 1]
    sems = rest[3 * _NB + 1:4 * _NB + 1]
    rows_v0 = rows[0]
    semg0 = semg[0]
    sid = lax.axis_index("s")
    wid = lax.axis_index("c") * 16 + sid
    rbase = sid * rg
    is_big = wid < extra
    nn = jnp.where(is_big, nn_max, nn_lo)
    nlo = pl.multiple_of(8 * (wid * base_oct + jnp.minimum(wid, extra)), 8)
    pltpu.sync_copy(ptr_hbm.at[pl.ds(nlo, _PTRBUF)], ptr_v)

    def zrow(r, carry):
      for k in range(d // 16):
        rows_v0[r, pl.ds(k * 16, 16)] = jnp.zeros((16,), jnp.float32)
      return carry
    lax.fori_loop(0, _CH, zrow, 0)
    for j in range(rg // _CH):
      pltpu.sync_copy(rows_v0, acc_sh.at[pl.ds(rbase + j * _CH, _CH)])

    elo = jnp.min(plsc.load_gather(ptr_v, [jnp.zeros((16,), jnp.int32)]))
    ehi = jnp.min(plsc.load_gather(ptr_v, [jnp.full((16,), nn, jnp.int32)]))
    c0 = lax.div(elo, _CH)
    c1 = lax.div(ehi + (_CH - 1), _CH)
    nch = c1 - c0
    ngrp = lax.div(nch + (_IB - 1), _IB)
    lanes = jnp.arange(16, dtype=jnp.int32)

    def seg_ids(c, out_ref):
      ebase = c * _CH
      for g in range(_CH // 16):
        e = ebase + g * 16 + lanes
        res = jnp.zeros((16,), jnp.int32)
        for stv in steps:
          cand = res + stv
          candc = jnp.minimum(cand, nn)
          pv = plsc.load_gather(ptr_v, [candc])
          ok = (cand <= nn) & (pv <= e)
          res = jnp.where(ok, cand, res)
        seg = rbase + jnp.where((e >= elo) & (e < ehi), res, nn_max)
        out_ref[pl.ds(g * 16, 16)] = seg

    def wait_scatter(b):
      pltpu.make_async_copy(rows[b], acc_sh.at[segc[b]], sems[b]).wait()

    nblk = idx2_hbm.shape[0]

    def grp(kb, carry):
      cb = c0 + kb * _IB
      cload = jnp.minimum(cb, nblk - _IB)
      sh = cb - cload
      pltpu.sync_copy(idx2_hbm.at[pl.ds(cload, _IB)], idxb_v)
      pltpu.async_copy(table_hbm.at[idxb_v.at[sh]], rows_v0, semg0)

      def triple(p, carry2):
        j0 = _NB * p
        for b in range(_NB):
          j = j0 + b
          c = cb + j
          bn = (b + 1) % _NB

          @pl.when((c - 2 >= c0) & (c - 2 < c1))
          def _(bn=bn):
            wait_scatter(bn)

          @pl.when((j + 1 < _IB) & (c + 1 < c1))
          def _(j=j, bn=bn):
            pltpu.async_copy(
                table_hbm.at[idxb_v.at[j + 1 + sh]], rows[bn], semg[bn])

          @pl.when(c < c1)
          def _(c=c, j=j, b=b):
            seg_ids(c, segc[b])
            pltpu.make_async_copy(
                table_hbm.at[idxb_v.at[j + sh]], rows[b], semg[b]).wait()
            pltpu.async_copy(rows[b], acc_sh.at[segc[b]], sems[b], add=True)
        return carry2
      lax.fori_loop(0, _IB // _NB, triple, 0)
      return carry
    lax.fori_loop(0, ngrp, grp, 0)

    t_end = c0 + ngrp * _IB

    @pl.when((t_end - 2 >= c0) & (t_end - 2 < c1))
    def _():
      wait_scatter(1)

    @pl.when((t_end - 1 >= c0) & (t_end - 1 < c1))
    def _():
      wait_scatter(2)

    pltpu.sync_copy(acc_sh.at[pl.ds(rbase, nn_lo)],
                    out_hbm.at[pl.ds(nlo, nn_lo)])

    @pl.when(is_big)
    def _():
      pltpu.sync_copy(acc_sh.at[pl.ds(rbase + nn_lo, 8)],
                      out_hbm.at[pl.ds(nlo + nn_lo, 8)])

  return seg_sum


_seg_sum_cache = {}


def _seg_sum(num_nodes, d):
  key = (num_nodes, d)
  if key not in _seg_sum_cache:
    _seg_sum_cache[key] = _make_seg_sum(num_nodes, d)
  return _seg_sum_cache[key]


def _mm_t(a, w):
  return lax.dot_general(a, w, (((1,), (1,)), ((), ())),
                         preferred_element_type=jnp.float32)


def _layer0_tc(agg0, dr0, x8, wl0, wr0, b0, scale, shift, wl1):
  bl = 800
  grid = _N1 // bl

  def body(agg_r, dr_r, x_r, wl0_r, wr0_r, b0_r, sc_r, sh_r, wl1_r,
           h_r, hw_r):
    z = _mm_t(agg_r[...] * dr_r[...], wl0_r[...])
    z = z + _mm_t(x_r[...], wr0_r[...])
    z = z + b0_r[...]
    h = jnp.maximum(z * sc_r[...] + sh_r[...], 0.0)
    h_r[...] = h
    hw_r[...] = _mm_t(h, wl1_r[...])

  full = lambda shape: pl.BlockSpec(shape, lambda i: (0, 0))
  return pl.pallas_call(
      body,
      grid=(grid,),
      in_specs=[
          pl.BlockSpec((bl, _D_H), lambda i: (i, 0)),
          pl.BlockSpec((bl, 1), lambda i: (i, 0)),
          pl.BlockSpec((bl, _D_IN), lambda i: (i, 0)),
          full((_D_H, _D_IN)),
          full((_D_H, _D_IN)),
          full((1, _D_H)),
          full((1, _D_H)),
          full((1, _D_H)),
          full((_D_OUT, _D_H)),
      ],
      out_specs=[
          pl.BlockSpec((bl, _D_H), lambda i: (i, 0)),
          pl.BlockSpec((bl, _D_OUT), lambda i: (i, 0)),
      ],
      out_shape=[
          jax.ShapeDtypeStruct((_N1, _D_H), jnp.float32),
          jax.ShapeDtypeStruct((_N1, _D_OUT), jnp.float32),
      ],
  )(agg0, dr0, x8, wl0, wr0, b0, scale, shift, wl1)


def _layer1_tc(agg1, dr1, h4, wr1, b1):
  bl = 400
  grid = _N0 // bl

  def body(agg_r, dr_r, h_r, wr1_r, b1_r, o_r):
    z = agg_r[...] * dr_r[...]
    z = z + _mm_t(h_r[...], wr1_r[...])
    z = z + b1_r[...]
    m = jnp.max(z, axis=-1, keepdims=True)
    ez = jnp.exp(z - m)
    o_r[...] = z - m - jnp.log(jnp.sum(ez, axis=-1, keepdims=True))

  full = lambda shape: pl.BlockSpec(shape, lambda i: (0, 0))
  return pl.pallas_call(
      body,
      grid=(grid,),
      in_specs=[
          pl.BlockSpec((bl, _D_OUT), lambda i: (i, 0)),
          pl.BlockSpec((bl, 1), lambda i: (i, 0)),
          pl.BlockSpec((bl, _D_H), lambda i: (i, 0)),
          full((_D_OUT, _D_H)),
          full((1, _D_OUT)),
      ],
      out_specs=pl.BlockSpec((bl, _D_OUT), lambda i: (i, 0)),
      out_shape=jax.ShapeDtypeStruct((_N0, _D_OUT), jnp.float32),
  )(agg1, dr1, h4, wr1, b1)


def kernel(x, ptr, idx, W_l0, b_l0, W_r0, gamma, beta, rmean, rvar,
           W_l1, b_l1, W_r1):
  ptr_pad = jnp.concatenate(
      [ptr, jnp.full((_PTRBUF,), jnp.iinfo(jnp.int32).max, jnp.int32)])
  idx2 = idx.reshape(_E // _CH, _CH)

  dr0 = 1.0 / jnp.maximum(ptr[1:_N1 + 1] - ptr[:_N1], 1).astype(jnp.float32)
  dr1 = 1.0 / jnp.maximum(ptr[1:_N0 + 1] - ptr[:_N0], 1).astype(jnp.float32)

  scale = gamma * lax.rsqrt(rvar + _EPS)
  shift = beta - rmean * scale

  agg0 = _seg_sum(_N1, _D_H)(x, ptr_pad, idx2)
  h, hw = _layer0_tc(agg0, dr0[:, None], x[:_N1], W_l0, W_r0,
                     b_l0[None, :], scale[None, :], shift[None, :], W_l1)
  agg1 = _seg_sum(_N0, _D_OUT)(hw, ptr_pad, idx2)
  return _layer1_tc(agg1, dr1[:, None], h[:_N0], W_r1, b_l1[None, :])
